# baseline (speedup 1.0000x reference)
import functools

import jax
import jax.numpy as jnp
from jax import lax
from jax.experimental import pallas as pl
from jax.experimental.pallas import tpu as pltpu
from jax.experimental.pallas import tpu_sc as plsc

N = 1024
E = 523776
D = 64
NW = 32
SC = 512
NSC = E // SC
ACC_ROWS = N + 8
ROWS_PER_SUB = N // 16
TRASH = N

_mesh = plsc.VectorSubcoreMesh(core_axis_name="c", subcore_axis_name="s")


def _wid():
    return lax.axis_index("s") * 2 + lax.axis_index("c")


def _span(wid):
    base = wid * (32 * SC)
    nsc = jnp.where(wid < 31, 32, 31)
    return base, nsc


@functools.partial(
    pl.kernel,
    out_type=jax.ShapeDtypeStruct((2, N, D), jnp.float32),
    mesh=_mesh,
    scratch_types=[
        pltpu.VMEM((SC, D), jnp.float32),
        pltpu.VMEM((SC,), jnp.int32),
        pltpu.VMEM((SC,), jnp.int32),
        pltpu.VMEM((SC,), jnp.float32),
    ] + [pltpu.VMEM((128,), jnp.int32) for _ in range(8)] + [
        pltpu.VMEM((ROWS_PER_SUB, D), jnp.float32),
        pltpu.VMEM_SHARED((ACC_ROWS, D), jnp.float32),
    ],
    compiler_params=pltpu.CompilerParams(use_tc_tiling_on_sc=False),
)
def _scatter_kernel(edge_hbm, idx1_hbm, idx2_hbm, mask_hbm, out_hbm,
                    rows_v, i1_v, i2_v, m_v,
                    ea0, ea1, ea2, ea3, eb0, eb1, eb2, eb3, z_v, acc_sh):
    e1_refs = [ea0, ea1, ea2, ea3]
    e2_refs = [eb0, eb1, eb2, eb3]
    c = lax.axis_index("c")
    s = lax.axis_index("s")
    wid = _wid()
    base, nsc = _span(wid)

    zvec = jnp.zeros((16,), jnp.float32)
    for r in range(ROWS_PER_SUB):
        for q in range(4):
            z_v[r, pl.ds(q * 16, 16)] = zvec
    pltpu.sync_copy(z_v, acc_sh.at[pl.ds(s * ROWS_PER_SUB, ROWS_PER_SUB)])
    plsc.subcore_barrier()

    def body(j, _):
        off = base + j * SC
        pltpu.sync_copy(edge_hbm.at[pl.ds(off, SC)], rows_v)
        pltpu.sync_copy(idx1_hbm.at[pl.ds(off, SC)], i1_v)
        pltpu.sync_copy(idx2_hbm.at[pl.ds(off, SC)], i2_v)
        pltpu.sync_copy(mask_hbm.at[pl.ds(off, SC)], m_v)
        for j2 in range(4):
            for k in range(8):
                g = j2 * 128 + k * 16
                keep = m_v[pl.ds(g, 16)] > 0.0
                e1_refs[j2][pl.ds(k * 16, 16)] = jnp.where(
                    keep, i1_v[pl.ds(g, 16)], TRASH)
                e2_refs[j2][pl.ds(k * 16, 16)] = jnp.where(
                    keep, i2_v[pl.ds(g, 16)], TRASH)
        for j2 in range(4):
            blk = rows_v.at[pl.ds(j2 * 128, 128)]
            pltpu.sync_copy(blk, acc_sh.at[e1_refs[j2]], add=True)
            pltpu.sync_copy(blk, acc_sh.at[e2_refs[j2]], add=True)
        return ()

    lax.fori_loop(0, nsc, body, ())
    plsc.subcore_barrier()
    pltpu.sync_copy(acc_sh.at[pl.ds(s * ROWS_PER_SUB, ROWS_PER_SUB)],
                    out_hbm.at[c].at[pl.ds(s * ROWS_PER_SUB, ROWS_PER_SUB)])


def _node_body(agg_ref, nf_ref, w_ref, b_ref, out_ref):
    h = jnp.tanh(
        jax.lax.dot(agg_ref[...], w_ref[...],
                    preferred_element_type=jnp.float32)
        + b_ref[0:1, :])
    out_ref[...] = nf_ref[...] + h


_node_call = pl.pallas_call(
    _node_body,
    out_shape=jax.ShapeDtypeStruct((N, D), jnp.float32),
)


@functools.partial(
    pl.kernel,
    out_type=jax.ShapeDtypeStruct((E, D), jnp.float32),
    mesh=_mesh,
    scratch_types=[
        pltpu.VMEM((N, D), jnp.float32),
        pltpu.VMEM((SC, D), jnp.float32),
        pltpu.VMEM((SC,), jnp.int32),
        pltpu.VMEM((SC,), jnp.int32),
        pltpu.VMEM((SC,), jnp.float32),
        pltpu.VMEM((D,), jnp.float32),
        pltpu.VMEM((D,), jnp.float32),
        pltpu.VMEM((D,), jnp.float32),
    ],
    compiler_params=pltpu.CompilerParams(use_tc_tiling_on_sc=False),
)
def _edge_kernel(node_hbm, edge_hbm, idx1_hbm, idx2_hbm, mask_hbm,
                 wg_hbm, ws_hbm, be_hbm, out_hbm,
                 table_v, ef_v, i1_v, i2_v, m_v, wg_v, ws_v, be_v):
    wid = _wid()
    base, nsc = _span(wid)

    pltpu.sync_copy(node_hbm, table_v)
    pltpu.sync_copy(wg_hbm, wg_v)
    pltpu.sync_copy(ws_hbm, ws_v)
    pltpu.sync_copy(be_hbm, be_v)
    wg = [wg_v[pl.ds(q * 16, 16)] for q in range(4)]
    ws = [ws_v[pl.ds(q * 16, 16)] for q in range(4)]
    be = [be_v[pl.ds(q * 16, 16)] for q in range(4)]

    def body(j, _):
        off = base + j * SC
        pltpu.sync_copy(idx1_hbm.at[pl.ds(off, SC)], i1_v)
        pltpu.sync_copy(idx2_hbm.at[pl.ds(off, SC)], i2_v)
        pltpu.sync_copy(mask_hbm.at[pl.ds(off, SC)], m_v)
        pltpu.sync_copy(edge_hbm.at[pl.ds(off, SC)], ef_v)

        def gbody(g, _):
            i1_16 = i1_v[pl.ds(g * 16, 16)]
            i2_16 = i2_v[pl.ds(g * 16, 16)]
            m16 = m_v[pl.ds(g * 16, 16)]
            for i in range(16):
                e = g * 16 + i
                n1 = i1_16[i]
                n2 = i2_16[i]
                m = m16[i]
                for q in range(4):
                    sl = pl.ds(q * 16, 16)
                    a = table_v[n1, sl] + table_v[n2, sl]
                    z = a * wg[q] + ef_v[e, sl] * ws[q] + be[q]
                    t = 1.0 - 2.0 / (jnp.exp(2.0 * z) + 1.0)
                    ef_v[e, sl] = t * m
            return ()

        lax.fori_loop(0, SC // 16, gbody, ())
        pltpu.sync_copy(ef_v, out_hbm.at[pl.ds(off, SC)])
        return ()

    lax.fori_loop(0, nsc, body, ())


def kernel(node_feat, edge_feat, x_indices1, x_indices2, mask_valid,
           W_e2n, b_e2n, w_gate, w_self, b_edge):
    edge2d = edge_feat.reshape(E, D)
    mask1d = mask_valid.reshape(E)
    node2d = node_feat.reshape(N, D)

    partials = _scatter_kernel(edge2d, x_indices1, x_indices2, mask1d)
    agg = partials[0] + partials[1]
    node_out2d = _node_call(agg, node2d, W_e2n,
                            jnp.tile(b_e2n.reshape(1, D), (8, 1)))
    edge_out2d = _edge_kernel(node_out2d, edge2d, x_indices1, x_indices2,
                              mask1d, w_gate, w_self, b_edge)
    return (node_out2d.reshape(1, N, D), edge_out2d.reshape(1, E, D))


# speedup vs baseline: 1.2906x; 1.2906x over previous
import functools

import jax
import jax.numpy as jnp
from jax import lax
from jax.experimental import pallas as pl
from jax.experimental.pallas import tpu as pltpu
from jax.experimental.pallas import tpu_sc as plsc

N = 1024
E = 523776
D = 64
NW = 32
SC = 512
NSC = E // SC
ACC_ROWS = N + 8
ROWS_PER_SUB = N // 16
TRASH = N

_mesh = plsc.VectorSubcoreMesh(core_axis_name="c", subcore_axis_name="s")


def _wid():
    return lax.axis_index("s") * 2 + lax.axis_index("c")


def _span(wid):
    base = wid * (32 * SC)
    nsc = jnp.where(wid < 31, 32, 31)
    return base, nsc


@functools.partial(
    pl.kernel,
    out_type=jax.ShapeDtypeStruct((2, N, D), jnp.float32),
    mesh=_mesh,
    scratch_types=[
        pltpu.VMEM((SC, D), jnp.float32),
        pltpu.VMEM((SC,), jnp.int32),
        pltpu.VMEM((SC,), jnp.int32),
        pltpu.VMEM((SC,), jnp.float32),
    ] + [pltpu.VMEM((128,), jnp.int32) for _ in range(8)] + [
        pltpu.VMEM((ROWS_PER_SUB, D), jnp.float32),
        pltpu.VMEM_SHARED((ACC_ROWS, D), jnp.float32),
    ],
    compiler_params=pltpu.CompilerParams(use_tc_tiling_on_sc=False),
)
def _scatter_kernel(edge_hbm, idx1_hbm, idx2_hbm, mask_hbm, out_hbm,
                    rows_v, i1_v, i2_v, m_v,
                    ea0, ea1, ea2, ea3, eb0, eb1, eb2, eb3, z_v, acc_sh):
    e1_refs = [ea0, ea1, ea2, ea3]
    e2_refs = [eb0, eb1, eb2, eb3]
    c = lax.axis_index("c")
    s = lax.axis_index("s")
    wid = _wid()
    base, nsc = _span(wid)

    zvec = jnp.zeros((16,), jnp.float32)
    for r in range(ROWS_PER_SUB):
        for q in range(4):
            z_v[r, pl.ds(q * 16, 16)] = zvec
    pltpu.sync_copy(z_v, acc_sh.at[pl.ds(s * ROWS_PER_SUB, ROWS_PER_SUB)])
    plsc.subcore_barrier()

    def body(j, _):
        off = base + j * SC
        pltpu.sync_copy(edge_hbm.at[pl.ds(off, SC)], rows_v)
        pltpu.sync_copy(idx1_hbm.at[pl.ds(off, SC)], i1_v)
        pltpu.sync_copy(idx2_hbm.at[pl.ds(off, SC)], i2_v)
        pltpu.sync_copy(mask_hbm.at[pl.ds(off, SC)], m_v)
        for j2 in range(4):
            for k in range(8):
                g = j2 * 128 + k * 16
                keep = m_v[pl.ds(g, 16)] > 0.0
                e1_refs[j2][pl.ds(k * 16, 16)] = jnp.where(
                    keep, i1_v[pl.ds(g, 16)], TRASH)
                e2_refs[j2][pl.ds(k * 16, 16)] = jnp.where(
                    keep, i2_v[pl.ds(g, 16)], TRASH)
        for j2 in range(4):
            blk = rows_v.at[pl.ds(j2 * 128, 128)]
            pltpu.sync_copy(blk, acc_sh.at[e1_refs[j2]], add=True)
            pltpu.sync_copy(blk, acc_sh.at[e2_refs[j2]], add=True)
        return ()

    lax.fori_loop(0, nsc, body, ())
    plsc.subcore_barrier()
    pltpu.sync_copy(acc_sh.at[pl.ds(s * ROWS_PER_SUB, ROWS_PER_SUB)],
                    out_hbm.at[c].at[pl.ds(s * ROWS_PER_SUB, ROWS_PER_SUB)])


def _node_body(agg_ref, nf_ref, w_ref, b_ref, out_ref):
    h = jnp.tanh(
        jax.lax.dot(agg_ref[...], w_ref[...],
                    preferred_element_type=jnp.float32)
        + b_ref[0:1, :])
    out_ref[...] = nf_ref[...] + h


_node_call = pl.pallas_call(
    _node_body,
    out_shape=jax.ShapeDtypeStruct((N, D), jnp.float32),
)


CH = 128


def _espan(wid):
    base = wid * (128 * CH) - jnp.maximum(wid - 28, 0) * CH
    nch = jnp.where(wid < 28, 128, 127)
    return base, nch


@functools.partial(
    pl.kernel,
    out_type=jax.ShapeDtypeStruct((E, D), jnp.float32),
    mesh=_mesh,
    scratch_types=[
        pltpu.VMEM((2, CH, D), jnp.float32),
        pltpu.VMEM((2, CH, D), jnp.float32),
        pltpu.VMEM((2, CH, D), jnp.float32),
        pltpu.VMEM((2, CH, D), jnp.float32),
        pltpu.VMEM((2, CH), jnp.int32),
        pltpu.VMEM((2, CH), jnp.int32),
        pltpu.VMEM((2, CH), jnp.float32),
        pltpu.VMEM((D,), jnp.float32),
        pltpu.VMEM((D,), jnp.float32),
        pltpu.VMEM((D,), jnp.float32),
        pltpu.SemaphoreType.DMA,
        pltpu.SemaphoreType.DMA,
        pltpu.SemaphoreType.DMA,
        pltpu.SemaphoreType.DMA,
        pltpu.SemaphoreType.DMA,
        pltpu.SemaphoreType.DMA,
    ],
    compiler_params=pltpu.CompilerParams(use_tc_tiling_on_sc=False),
)
def _edge_kernel(node_hbm, edge_hbm, idx1_hbm, idx2_hbm, mask_hbm,
                 wg_hbm, ws_hbm, be_hbm, out_hbm,
                 ef_v, n1_v, n2_v, eo_v, i1_v, i2_v, m_v, wg_v, ws_v, be_v,
                 sl0, sl1, sg0, sg1, ss0, ss1):
    wid = _wid()
    base, nch = _espan(wid)
    sem_l = [sl0, sl1]
    sem_g = [sg0, sg1]
    sem_s = [ss0, ss1]

    def off(j):
        return base + jnp.minimum(j, nch - 1) * CH

    def load_set(j, b):
        o = off(j)
        return [
            (edge_hbm.at[pl.ds(o, CH)], ef_v.at[b], sem_l[b]),
            (idx1_hbm.at[pl.ds(o, CH)], i1_v.at[b], sem_l[b]),
            (idx2_hbm.at[pl.ds(o, CH)], i2_v.at[b], sem_l[b]),
            (mask_hbm.at[pl.ds(o, CH)], m_v.at[b], sem_l[b]),
        ]

    def gather_set(b):
        return [
            (node_hbm.at[i1_v.at[b]], n1_v.at[b], sem_g[b]),
            (node_hbm.at[i2_v.at[b]], n2_v.at[b], sem_g[b]),
        ]

    def store_desc(j, b):
        return (eo_v.at[b], out_hbm.at[pl.ds(off(j), CH)], sem_s[b])

    def issue(descs):
        for s_, d_, m_ in descs:
            pltpu.async_copy(s_, d_, m_)

    def drain(descs):
        for s_, d_, m_ in descs:
            pltpu.make_async_copy(s_, d_, m_).wait()

    def compute(b):
        def gbody(g, _):
            m16 = m_v[b, pl.ds(g * 16, 16)]
            for i in range(16):
                e = g * 16 + i
                m = m16[i]
                for q in range(4):
                    sl = pl.ds(q * 16, 16)
                    a = n1_v[b, e, sl] + n2_v[b, e, sl]
                    z = a * wg[q] + ef_v[b, e, sl] * ws[q] + be[q]
                    t = 1.0 - 2.0 / (jnp.exp(2.0 * z) + 1.0)
                    eo_v[b, e, sl] = t * m
            return ()

        lax.fori_loop(0, CH // 16, gbody, ())

    issue(load_set(0, 0))
    issue(load_set(1, 1))
    pltpu.sync_copy(wg_hbm, wg_v)
    pltpu.sync_copy(ws_hbm, ws_v)
    pltpu.sync_copy(be_hbm, be_v)
    wg = [wg_v[pl.ds(q * 16, 16)] for q in range(4)]
    ws = [ws_v[pl.ds(q * 16, 16)] for q in range(4)]
    be = [be_v[pl.ds(q * 16, 16)] for q in range(4)]
    drain(load_set(0, 0))
    issue(gather_set(0))

    for j, b in ((0, 0), (1, 1)):
        drain(gather_set(b))
        compute(b)
        issue([store_desc(j, b)])
        issue(load_set(j + 2, b))
        drain(load_set(j + 1, 1 - b))
        issue(gather_set(1 - b))

    def step(s, _):
        for b in (0, 1):
            j = 2 * s + b
            drain([store_desc(j - 2, b)])
            drain(gather_set(b))
            compute(b)
            issue([store_desc(j, b)])
            issue(load_set(j + 2, b))
            drain(load_set(j + 1, 1 - b))
            issue(gather_set(1 - b))
        return ()

    lax.fori_loop(1, 64, step, ())

    drain([store_desc(126, 0)])
    drain([store_desc(127, 1)])
    drain(load_set(129, 1))
    drain(gather_set(0))


def kernel(node_feat, edge_feat, x_indices1, x_indices2, mask_valid,
           W_e2n, b_e2n, w_gate, w_self, b_edge):
    edge2d = edge_feat.reshape(E, D)
    mask1d = mask_valid.reshape(E)
    node2d = node_feat.reshape(N, D)

    partials = _scatter_kernel(edge2d, x_indices1, x_indices2, mask1d)
    agg = partials[0] + partials[1]
    node_out2d = _node_call(agg, node2d, W_e2n,
                            jnp.tile(b_e2n.reshape(1, D), (8, 1)))
    edge_out2d = _edge_kernel(node_out2d, edge2d, x_indices1, x_indices2,
                              mask1d, w_gate, w_self, b_edge)
    return (node_out2d.reshape(1, N, D), edge_out2d.reshape(1, E, D))


# speedup vs baseline: 1.3353x; 1.0346x over previous
import functools

import jax
import jax.numpy as jnp
from jax import lax
from jax.experimental import pallas as pl
from jax.experimental.pallas import tpu as pltpu
from jax.experimental.pallas import tpu_sc as plsc

N = 1024
E = 523776
D = 64
NW = 32
SC = 512
NSC = E // SC
ACC_ROWS = N + 8
ROWS_PER_SUB = N // 16
TRASH = N

_mesh = plsc.VectorSubcoreMesh(core_axis_name="c", subcore_axis_name="s")


def _wid():
    return lax.axis_index("s") * 2 + lax.axis_index("c")


def _span(wid):
    base = wid * (32 * SC)
    nsc = jnp.where(wid < 31, 32, 31)
    return base, nsc


@functools.partial(
    pl.kernel,
    out_type=jax.ShapeDtypeStruct((2, N, D), jnp.float32),
    mesh=_mesh,
    scratch_types=[
        pltpu.VMEM((SC, D), jnp.float32),
        pltpu.VMEM((SC,), jnp.int32),
        pltpu.VMEM((SC,), jnp.int32),
        pltpu.VMEM((SC,), jnp.float32),
    ] + [pltpu.VMEM((128,), jnp.int32) for _ in range(8)] + [
        pltpu.VMEM((ROWS_PER_SUB, D), jnp.float32),
        pltpu.VMEM_SHARED((ACC_ROWS, D), jnp.float32),
    ],
    compiler_params=pltpu.CompilerParams(use_tc_tiling_on_sc=False),
)
def _scatter_kernel(edge_hbm, idx1_hbm, idx2_hbm, mask_hbm, out_hbm,
                    rows_v, i1_v, i2_v, m_v,
                    ea0, ea1, ea2, ea3, eb0, eb1, eb2, eb3, z_v, acc_sh):
    e1_refs = [ea0, ea1, ea2, ea3]
    e2_refs = [eb0, eb1, eb2, eb3]
    c = lax.axis_index("c")
    s = lax.axis_index("s")
    wid = _wid()
    base, nsc = _span(wid)

    zvec = jnp.zeros((16,), jnp.float32)
    for r in range(ROWS_PER_SUB):
        for q in range(4):
            z_v[r, pl.ds(q * 16, 16)] = zvec
    pltpu.sync_copy(z_v, acc_sh.at[pl.ds(s * ROWS_PER_SUB, ROWS_PER_SUB)])
    plsc.subcore_barrier()

    def body(j, _):
        off = base + j * SC
        pltpu.sync_copy(edge_hbm.at[pl.ds(off, SC)], rows_v)
        pltpu.sync_copy(idx1_hbm.at[pl.ds(off, SC)], i1_v)
        pltpu.sync_copy(idx2_hbm.at[pl.ds(off, SC)], i2_v)
        pltpu.sync_copy(mask_hbm.at[pl.ds(off, SC)], m_v)
        for j2 in range(4):
            for k in range(8):
                g = j2 * 128 + k * 16
                keep = m_v[pl.ds(g, 16)] > 0.0
                e1_refs[j2][pl.ds(k * 16, 16)] = jnp.where(
                    keep, i1_v[pl.ds(g, 16)], TRASH)
                e2_refs[j2][pl.ds(k * 16, 16)] = jnp.where(
                    keep, i2_v[pl.ds(g, 16)], TRASH)
        for j2 in range(4):
            blk = rows_v.at[pl.ds(j2 * 128, 128)]
            pltpu.sync_copy(blk, acc_sh.at[e1_refs[j2]], add=True)
            pltpu.sync_copy(blk, acc_sh.at[e2_refs[j2]], add=True)
        return ()

    lax.fori_loop(0, nsc, body, ())
    plsc.subcore_barrier()
    pltpu.sync_copy(acc_sh.at[pl.ds(s * ROWS_PER_SUB, ROWS_PER_SUB)],
                    out_hbm.at[c].at[pl.ds(s * ROWS_PER_SUB, ROWS_PER_SUB)])


def _node_body(agg_ref, nf_ref, w_ref, b_ref, out_ref):
    h = jnp.tanh(
        jax.lax.dot(agg_ref[...], w_ref[...],
                    preferred_element_type=jnp.float32)
        + b_ref[0:1, :])
    out_ref[...] = nf_ref[...] + h


_node_call = pl.pallas_call(
    _node_body,
    out_shape=jax.ShapeDtypeStruct((N, D), jnp.float32),
)


CH = 128


def _espan(wid):
    base = wid * (128 * CH) - jnp.maximum(wid - 28, 0) * CH
    nch = jnp.where(wid < 28, 128, 127)
    return base, nch


@functools.partial(
    pl.kernel,
    out_type=jax.ShapeDtypeStruct((E, D), jnp.float32),
    mesh=_mesh,
    scratch_types=[
        pltpu.VMEM((2, CH, D), jnp.float32),
        pltpu.VMEM((2, CH, D), jnp.float32),
        pltpu.VMEM((2, CH, D), jnp.float32),
        pltpu.VMEM((2, CH, D), jnp.float32),
        pltpu.VMEM((2, CH), jnp.int32),
        pltpu.VMEM((2, CH), jnp.int32),
        pltpu.VMEM((2, CH), jnp.float32),
        pltpu.VMEM((D,), jnp.float32),
        pltpu.VMEM((D,), jnp.float32),
        pltpu.VMEM((D,), jnp.float32),
        pltpu.SemaphoreType.DMA,
        pltpu.SemaphoreType.DMA,
        pltpu.SemaphoreType.DMA,
        pltpu.SemaphoreType.DMA,
        pltpu.SemaphoreType.DMA,
        pltpu.SemaphoreType.DMA,
    ],
    compiler_params=pltpu.CompilerParams(use_tc_tiling_on_sc=False),
)
def _edge_kernel(node_hbm, edge_hbm, idx1_hbm, idx2_hbm, mask_hbm,
                 wg_hbm, ws_hbm, be_hbm, out_hbm,
                 ef_v, n1_v, n2_v, eo_v, i1_v, i2_v, m_v, wg_v, ws_v, be_v,
                 sl0, sl1, sg0, sg1, ss0, ss1):
    wid = _wid()
    base, nch = _espan(wid)
    sem_l = [sl0, sl1]
    sem_g = [sg0, sg1]
    sem_s = [ss0, ss1]

    def off(j):
        return base + jnp.minimum(j, nch - 1) * CH

    def load_set(j, b):
        o = off(j)
        return [
            (edge_hbm.at[pl.ds(o, CH)], ef_v.at[b], sem_l[b]),
            (idx1_hbm.at[pl.ds(o, CH)], i1_v.at[b], sem_l[b]),
            (idx2_hbm.at[pl.ds(o, CH)], i2_v.at[b], sem_l[b]),
            (mask_hbm.at[pl.ds(o, CH)], m_v.at[b], sem_l[b]),
        ]

    def gather_set(b):
        return [
            (node_hbm.at[i1_v.at[b]], n1_v.at[b], sem_g[b]),
            (node_hbm.at[i2_v.at[b]], n2_v.at[b], sem_g[b]),
        ]

    def store_desc(j, b):
        return (eo_v.at[b], out_hbm.at[pl.ds(off(j), CH)], sem_s[b])

    def issue(descs):
        for s_, d_, m_ in descs:
            pltpu.async_copy(s_, d_, m_)

    def drain(descs):
        for s_, d_, m_ in descs:
            pltpu.make_async_copy(s_, d_, m_).wait()

    def compute(b):
        def gbody(g, _):
            m16 = m_v[b, pl.ds(g * 16, 16)]
            for i in range(16):
                e = g * 16 + i
                m = m16[i]
                for q in range(4):
                    sl = pl.ds(q * 16, 16)
                    a = n1_v[b, e, sl] + n2_v[b, e, sl]
                    z = a * wg[q] + ef_v[b, e, sl] * ws[q] + be[q]
                    t = 1.0 - 2.0 / (jnp.exp(2.0 * z) + 1.0)
                    eo_v[b, e, sl] = t * m
            return ()

        lax.fori_loop(0, CH // 16, gbody, ())

    issue(load_set(0, 0))
    issue(load_set(1, 1))
    pltpu.sync_copy(wg_hbm, wg_v)
    pltpu.sync_copy(ws_hbm, ws_v)
    pltpu.sync_copy(be_hbm, be_v)
    wg = [wg_v[pl.ds(q * 16, 16)] for q in range(4)]
    ws = [ws_v[pl.ds(q * 16, 16)] for q in range(4)]
    be = [be_v[pl.ds(q * 16, 16)] for q in range(4)]
    drain(load_set(0, 0))
    issue(gather_set(0))

    for j, b in ((0, 0), (1, 1)):
        drain(gather_set(b))
        drain(load_set(j + 1, 1 - b))
        issue(gather_set(1 - b))
        compute(b)
        issue([store_desc(j, b)])
        issue(load_set(j + 2, b))

    def step(s, _):
        for b in (0, 1):
            j = 2 * s + b
            drain([store_desc(j - 2, b)])
            drain(gather_set(b))
            drain(load_set(j + 1, 1 - b))
            issue(gather_set(1 - b))
            compute(b)
            issue([store_desc(j, b)])
            issue(load_set(j + 2, b))
        return ()

    lax.fori_loop(1, 64, step, ())

    drain([store_desc(126, 0)])
    drain([store_desc(127, 1)])
    drain(load_set(129, 1))
    drain(gather_set(0))


def kernel(node_feat, edge_feat, x_indices1, x_indices2, mask_valid,
           W_e2n, b_e2n, w_gate, w_self, b_edge):
    edge2d = edge_feat.reshape(E, D)
    mask1d = mask_valid.reshape(E)
    node2d = node_feat.reshape(N, D)

    partials = _scatter_kernel(edge2d, x_indices1, x_indices2, mask1d)
    agg = partials[0] + partials[1]
    node_out2d = _node_call(agg, node2d, W_e2n,
                            jnp.tile(b_e2n.reshape(1, D), (8, 1)))
    edge_out2d = _edge_kernel(node_out2d, edge2d, x_indices1, x_indices2,
                              mask1d, w_gate, w_self, b_edge)
    return (node_out2d.reshape(1, N, D), edge_out2d.reshape(1, E, D))


# speedup vs baseline: 1.6607x; 1.2437x over previous
import functools

import jax
import jax.numpy as jnp
from jax import lax
from jax.experimental import pallas as pl
from jax.experimental.pallas import tpu as pltpu
from jax.experimental.pallas import tpu_sc as plsc

N = 1024
E = 523776
D = 64
NW = 32
SC = 512
NSC = E // SC
ACC_ROWS = N + 8
ROWS_PER_SUB = N // 16
TRASH = N

_mesh = plsc.VectorSubcoreMesh(core_axis_name="c", subcore_axis_name="s")


def _wid():
    return lax.axis_index("s") * 2 + lax.axis_index("c")


def _span(wid):
    base = wid * (32 * SC)
    nsc = jnp.where(wid < 31, 32, 31)
    return base, nsc


@functools.partial(
    pl.kernel,
    out_type=jax.ShapeDtypeStruct((2, N, D), jnp.float32),
    mesh=_mesh,
    scratch_types=[
        pltpu.VMEM((SC, D), jnp.float32),
        pltpu.VMEM((SC,), jnp.int32),
        pltpu.VMEM((SC,), jnp.int32),
        pltpu.VMEM((SC,), jnp.float32),
        pltpu.VMEM((SC,), jnp.int32),
        pltpu.VMEM((SC,), jnp.int32),
        pltpu.VMEM((ROWS_PER_SUB, D), jnp.float32),
        pltpu.VMEM_SHARED((ACC_ROWS, D), jnp.float32),
    ],
    compiler_params=pltpu.CompilerParams(use_tc_tiling_on_sc=False),
)
def _scatter_kernel(edge_hbm, idx1_hbm, idx2_hbm, mask_hbm, out_hbm,
                    rows_v, i1_v, i2_v, m_v, e1_v, e2_v, z_v, acc_sh):
    c = lax.axis_index("c")
    s = lax.axis_index("s")
    wid = _wid()
    base, nsc = _span(wid)

    zvec = jnp.zeros((16,), jnp.float32)
    for r in range(ROWS_PER_SUB):
        for q in range(4):
            z_v[r, pl.ds(q * 16, 16)] = zvec
    pltpu.sync_copy(z_v, acc_sh.at[pl.ds(s * ROWS_PER_SUB, ROWS_PER_SUB)])
    plsc.subcore_barrier()

    def body(j, _):
        off = base + j * SC
        pltpu.sync_copy(edge_hbm.at[pl.ds(off, SC)], rows_v)
        pltpu.sync_copy(idx1_hbm.at[pl.ds(off, SC)], i1_v)
        pltpu.sync_copy(idx2_hbm.at[pl.ds(off, SC)], i2_v)
        pltpu.sync_copy(mask_hbm.at[pl.ds(off, SC)], m_v)
        for g in range(32):
            keep = m_v[pl.ds(g * 16, 16)] > 0.0
            e1_v[pl.ds(g * 16, 16)] = jnp.where(
                keep, i1_v[pl.ds(g * 16, 16)], TRASH)
            e2_v[pl.ds(g * 16, 16)] = jnp.where(
                keep, i2_v[pl.ds(g * 16, 16)], TRASH)
        pltpu.sync_copy(rows_v, acc_sh.at[e1_v], add=True)
        pltpu.sync_copy(rows_v, acc_sh.at[e2_v], add=True)
        return ()

    lax.fori_loop(0, nsc, body, ())
    plsc.subcore_barrier()
    pltpu.sync_copy(acc_sh.at[pl.ds(s * ROWS_PER_SUB, ROWS_PER_SUB)],
                    out_hbm.at[c].at[pl.ds(s * ROWS_PER_SUB, ROWS_PER_SUB)])


def _node_body(agg_ref, nf_ref, w_ref, b_ref, out_ref):
    h = jnp.tanh(
        jax.lax.dot(agg_ref[...], w_ref[...],
                    preferred_element_type=jnp.float32)
        + b_ref[0:1, :])
    out_ref[...] = nf_ref[...] + h


_node_call = pl.pallas_call(
    _node_body,
    out_shape=jax.ShapeDtypeStruct((N, D), jnp.float32),
)


@functools.partial(
    pl.kernel,
    out_type=jax.ShapeDtypeStruct((E, D), jnp.float32),
    mesh=_mesh,
    scratch_types=[
        pltpu.VMEM((SC, D), jnp.float32),
        pltpu.VMEM((SC, D), jnp.float32),
        pltpu.VMEM((SC, D), jnp.float32),
        pltpu.VMEM((SC,), jnp.int32),
        pltpu.VMEM((SC,), jnp.int32),
        pltpu.VMEM((SC,), jnp.float32),
        pltpu.VMEM((D,), jnp.float32),
        pltpu.VMEM((D,), jnp.float32),
        pltpu.VMEM((D,), jnp.float32),
        pltpu.SemaphoreType.DMA,
    ],
    compiler_params=pltpu.CompilerParams(use_tc_tiling_on_sc=False),
)
def _edge_kernel(node_hbm, edge_hbm, idx1_hbm, idx2_hbm, mask_hbm,
                 wg_hbm, ws_hbm, be_hbm, out_hbm,
                 ef_v, n1_v, n2_v, i1_v, i2_v, m_v, wg_v, ws_v, be_v, sem):
    wid = _wid()
    base, nsc = _span(wid)

    pltpu.sync_copy(wg_hbm, wg_v)
    pltpu.sync_copy(ws_hbm, ws_v)
    pltpu.sync_copy(be_hbm, be_v)
    wg = [wg_v[pl.ds(q * 16, 16)] for q in range(4)]
    ws = [ws_v[pl.ds(q * 16, 16)] for q in range(4)]
    be = [be_v[pl.ds(q * 16, 16)] for q in range(4)]

    def body(j, _):
        off = base + j * SC
        pltpu.sync_copy(idx1_hbm.at[pl.ds(off, SC)], i1_v)
        pltpu.sync_copy(idx2_hbm.at[pl.ds(off, SC)], i2_v)
        pltpu.sync_copy(mask_hbm.at[pl.ds(off, SC)], m_v)
        pltpu.sync_copy(edge_hbm.at[pl.ds(off, SC)], ef_v)
        h1 = pltpu.async_copy(node_hbm.at[i1_v], n1_v, sem)
        h2 = pltpu.async_copy(node_hbm.at[i2_v], n2_v, sem)
        h1.wait()
        h2.wait()

        def gbody(g, _):
            m16 = m_v[pl.ds(g * 16, 16)]
            for i in range(16):
                e = g * 16 + i
                m = m16[i]
                for q in range(4):
                    sl = pl.ds(q * 16, 16)
                    a = n1_v[e, sl] + n2_v[e, sl]
                    z = a * wg[q] + ef_v[e, sl] * ws[q] + be[q]
                    t = 1.0 - 2.0 / (jnp.exp(2.0 * z) + 1.0)
                    ef_v[e, sl] = t * m
            return ()

        lax.fori_loop(0, SC // 16, gbody, ())
        pltpu.sync_copy(ef_v, out_hbm.at[pl.ds(off, SC)])
        return ()

    lax.fori_loop(0, nsc, body, ())


def kernel(node_feat, edge_feat, x_indices1, x_indices2, mask_valid,
           W_e2n, b_e2n, w_gate, w_self, b_edge):
    edge2d = edge_feat.reshape(E, D)
    mask1d = mask_valid.reshape(E)
    node2d = node_feat.reshape(N, D)

    partials = _scatter_kernel(edge2d, x_indices1, x_indices2, mask1d)
    agg = partials[0] + partials[1]
    node_out2d = _node_call(agg, node2d, W_e2n,
                            jnp.tile(b_e2n.reshape(1, D), (8, 1)))
    edge_out2d = _edge_kernel(node_out2d, edge2d, x_indices1, x_indices2,
                              mask1d, w_gate, w_self, b_edge)
    return (node_out2d.reshape(1, N, D), edge_out2d.reshape(1, E, D))


# speedup vs baseline: 1.7016x; 1.0246x over previous
import functools

import jax
import jax.numpy as jnp
from jax import lax
from jax.experimental import pallas as pl
from jax.experimental.pallas import tpu as pltpu
from jax.experimental.pallas import tpu_sc as plsc

N = 1024
E = 523776
D = 64
NW = 32
SC = 512
NSC = E // SC
ACC_ROWS = N + 8
ROWS_PER_SUB = N // 16
TRASH = N

_mesh = plsc.VectorSubcoreMesh(core_axis_name="c", subcore_axis_name="s")


def _wid():
    return lax.axis_index("s") * 2 + lax.axis_index("c")


def _span(wid):
    base = wid * (32 * SC)
    nsc = jnp.where(wid < 31, 32, 31)
    return base, nsc


@functools.partial(
    pl.kernel,
    out_type=jax.ShapeDtypeStruct((2, N, D), jnp.float32),
    mesh=_mesh,
    scratch_types=[
        pltpu.VMEM((SC, D), jnp.float32),
        pltpu.VMEM((SC,), jnp.int32),
        pltpu.VMEM((SC,), jnp.int32),
        pltpu.VMEM((SC,), jnp.float32),
        pltpu.VMEM((SC,), jnp.int32),
        pltpu.VMEM((SC,), jnp.int32),
        pltpu.VMEM((ROWS_PER_SUB, D), jnp.float32),
        pltpu.VMEM_SHARED((ACC_ROWS, D), jnp.float32),
    ],
    compiler_params=pltpu.CompilerParams(use_tc_tiling_on_sc=False),
)
def _scatter_kernel(edge_hbm, idx1_hbm, idx2_hbm, mask_hbm, out_hbm,
                    rows_v, i1_v, i2_v, m_v, e1_v, e2_v, z_v, acc_sh):
    c = lax.axis_index("c")
    s = lax.axis_index("s")
    wid = _wid()
    base, nsc = _span(wid)

    zvec = jnp.zeros((16,), jnp.float32)
    for r in range(ROWS_PER_SUB):
        for q in range(4):
            z_v[r, pl.ds(q * 16, 16)] = zvec
    pltpu.sync_copy(z_v, acc_sh.at[pl.ds(s * ROWS_PER_SUB, ROWS_PER_SUB)])
    plsc.subcore_barrier()

    def body(j, _):
        off = base + j * SC
        pltpu.sync_copy(edge_hbm.at[pl.ds(off, SC)], rows_v)
        pltpu.sync_copy(idx1_hbm.at[pl.ds(off, SC)], i1_v)
        pltpu.sync_copy(idx2_hbm.at[pl.ds(off, SC)], i2_v)
        pltpu.sync_copy(mask_hbm.at[pl.ds(off, SC)], m_v)
        for g in range(32):
            keep = m_v[pl.ds(g * 16, 16)] > 0.0
            e1_v[pl.ds(g * 16, 16)] = jnp.where(
                keep, i1_v[pl.ds(g * 16, 16)], TRASH)
            e2_v[pl.ds(g * 16, 16)] = jnp.where(
                keep, i2_v[pl.ds(g * 16, 16)], TRASH)
        pltpu.sync_copy(rows_v, acc_sh.at[e1_v], add=True)
        pltpu.sync_copy(rows_v, acc_sh.at[e2_v], add=True)
        return ()

    lax.fori_loop(0, nsc, body, ())
    plsc.subcore_barrier()
    pltpu.sync_copy(acc_sh.at[pl.ds(s * ROWS_PER_SUB, ROWS_PER_SUB)],
                    out_hbm.at[c].at[pl.ds(s * ROWS_PER_SUB, ROWS_PER_SUB)])


def _node_body(agg_ref, nf_ref, w_ref, b_ref, out_ref):
    h = jnp.tanh(
        jax.lax.dot(agg_ref[...], w_ref[...],
                    preferred_element_type=jnp.float32)
        + b_ref[0:1, :])
    out_ref[...] = nf_ref[...] + h


_node_call = pl.pallas_call(
    _node_body,
    out_shape=jax.ShapeDtypeStruct((N, D), jnp.float32),
)


@functools.partial(
    pl.kernel,
    out_type=jax.ShapeDtypeStruct((E, D), jnp.float32),
    mesh=_mesh,
    scratch_types=[
        pltpu.VMEM((SC, D), jnp.float32),
        pltpu.VMEM((SC, D), jnp.float32),
        pltpu.VMEM((SC, D), jnp.float32),
        pltpu.VMEM((SC,), jnp.int32),
        pltpu.VMEM((SC,), jnp.int32),
        pltpu.VMEM((SC,), jnp.float32),
        pltpu.VMEM((D,), jnp.float32),
        pltpu.VMEM((D,), jnp.float32),
        pltpu.VMEM((D,), jnp.float32),
    ] + [pltpu.SemaphoreType.DMA for _ in range(8)],
    compiler_params=pltpu.CompilerParams(use_tc_tiling_on_sc=False),
)
def _edge_kernel(node_hbm, edge_hbm, idx1_hbm, idx2_hbm, mask_hbm,
                 wg_hbm, ws_hbm, be_hbm, out_hbm,
                 ef_v, n1_v, n2_v, i1_v, i2_v, m_v, wg_v, ws_v, be_v,
                 *sems):
    wid = _wid()
    base, nsc = _span(wid)

    pltpu.sync_copy(wg_hbm, wg_v)
    pltpu.sync_copy(ws_hbm, ws_v)
    pltpu.sync_copy(be_hbm, be_v)
    wg = [wg_v[pl.ds(q * 16, 16)] for q in range(4)]
    ws = [ws_v[pl.ds(q * 16, 16)] for q in range(4)]
    be = [be_v[pl.ds(q * 16, 16)] for q in range(4)]

    def body(j, _):
        off = base + j * SC
        pltpu.sync_copy(idx1_hbm.at[pl.ds(off, SC)], i1_v)
        pltpu.sync_copy(idx2_hbm.at[pl.ds(off, SC)], i2_v)
        pltpu.sync_copy(mask_hbm.at[pl.ds(off, SC)], m_v)
        pltpu.sync_copy(edge_hbm.at[pl.ds(off, SC)], ef_v)
        hs = []
        for t in range(4):
            sl = pl.ds(t * 128, 128)
            hs.append((pltpu.async_copy(node_hbm.at[i1_v.at[sl]],
                                        n1_v.at[sl], sems[2 * t]),
                       pltpu.async_copy(node_hbm.at[i2_v.at[sl]],
                                        n2_v.at[sl], sems[2 * t + 1])))

        def gbody(g, _):
            m16 = m_v[pl.ds(g * 16, 16)]
            for i in range(16):
                e = g * 16 + i
                m = m16[i]
                for q in range(4):
                    sl = pl.ds(q * 16, 16)
                    a = n1_v[e, sl] + n2_v[e, sl]
                    z = a * wg[q] + ef_v[e, sl] * ws[q] + be[q]
                    t = 1.0 - 2.0 / (jnp.exp(2.0 * z) + 1.0)
                    ef_v[e, sl] = t * m
            return ()

        for t in range(4):
            h1, h2 = hs[t]
            h1.wait()
            h2.wait()
            lax.fori_loop(8 * t, 8 * (t + 1), gbody, ())
        pltpu.sync_copy(ef_v, out_hbm.at[pl.ds(off, SC)])
        return ()

    lax.fori_loop(0, nsc, body, ())


def kernel(node_feat, edge_feat, x_indices1, x_indices2, mask_valid,
           W_e2n, b_e2n, w_gate, w_self, b_edge):
    edge2d = edge_feat.reshape(E, D)
    mask1d = mask_valid.reshape(E)
    node2d = node_feat.reshape(N, D)

    partials = _scatter_kernel(edge2d, x_indices1, x_indices2, mask1d)
    agg = partials[0] + partials[1]
    node_out2d = _node_call(agg, node2d, W_e2n,
                            jnp.tile(b_e2n.reshape(1, D), (8, 1)))
    edge_out2d = _edge_kernel(node_out2d, edge2d, x_indices1, x_indices2,
                              mask1d, w_gate, w_self, b_edge)
    return (node_out2d.reshape(1, N, D), edge_out2d.reshape(1, E, D))
